# Initial kernel scaffold; baseline (speedup 1.0000x reference)
import jax
import jax.numpy as jnp
from jax.experimental import pallas as pl


def kernel(input_seq, weights):
    raise NotImplementedError("write your pallas kernel here")


# speedup vs baseline: 1.4588x; 1.4588x over previous
import functools

import jax
import jax.numpy as jnp
from jax import lax
from jax.experimental import pallas as pl
from jax.experimental.pallas import tpu as pltpu
from jax.experimental.pallas import tpu_sc as plsc

NUM_EMB = 1000000
D = 32
BATCH = 4096
HIST = 200
TOTAL = BATCH * HIST

NC = 2
NS = 16
NW = NC * NS
PER_W = TOTAL // NW

G = 128
CHUNK = 1024
N_G = CHUNK // G
N_IT = PER_W // CHUNK
IDX_ROWS_W = PER_W // G


def _build():
    mesh = plsc.VectorSubcoreMesh(core_axis_name="c", subcore_axis_name="s")

    @functools.partial(
        pl.kernel,
        mesh=mesh,
        out_type=jax.ShapeDtypeStruct((TOTAL, D), jnp.float32),
        scratch_types=[
            pltpu.VMEM((N_G, G), jnp.int32),
            pltpu.VMEM((CHUNK, D), jnp.float32),
            pltpu.SemaphoreType.DMA,
        ],
        compiler_params=pltpu.CompilerParams(use_tc_tiling_on_sc=False),
    )
    def body(idx_hbm, table_hbm, out_hbm, idx_v, rows_v, sem):
        wid = lax.axis_index("s") * NC + lax.axis_index("c")
        row_base = wid * PER_W
        idx_row_base = wid * IDX_ROWS_W

        def it(g, carry):
            pltpu.sync_copy(idx_hbm.at[pl.ds(idx_row_base + g * N_G, N_G)],
                            idx_v)
            handles = [
                pltpu.async_copy(table_hbm.at[idx_v.at[j]],
                                 rows_v.at[pl.ds(j * G, G)],
                                 sem)
                for j in range(N_G)
            ]
            for h in handles:
                h.wait()
            pltpu.sync_copy(rows_v,
                            out_hbm.at[pl.ds(row_base + g * CHUNK, CHUNK)])
            return carry

        lax.fori_loop(0, N_IT, it, 0)

    return body


_gather_kernel = _build()


def kernel(input_seq, weights):
    idx = input_seq.reshape(TOTAL // G, G)
    out = _gather_kernel(idx, weights)
    return out.reshape(BATCH, HIST, D)


# speedup vs baseline: 1.4977x; 1.0267x over previous
import functools

import jax
import jax.numpy as jnp
from jax import lax
from jax.experimental import pallas as pl
from jax.experimental.pallas import tpu as pltpu
from jax.experimental.pallas import tpu_sc as plsc

NUM_EMB = 1000000
D = 32
BATCH = 4096
HIST = 200
TOTAL = BATCH * HIST

NC = 2
NS = 16
NW = NC * NS
PER_W = TOTAL // NW

G = 128
CHUNK = 1280
N_G = CHUNK // G
N_IT = PER_W // CHUNK
IDX_ROWS_W = PER_W // G


def _build():
    mesh = plsc.VectorSubcoreMesh(core_axis_name="c", subcore_axis_name="s")

    @functools.partial(
        pl.kernel,
        mesh=mesh,
        out_type=jax.ShapeDtypeStruct((TOTAL, D), jnp.float32),
        scratch_types=[
            pltpu.VMEM((IDX_ROWS_W, G), jnp.int32),
            pltpu.VMEM((CHUNK, D), jnp.float32),
            pltpu.VMEM((CHUNK, D), jnp.float32),
            pltpu.SemaphoreType.DMA,
            pltpu.SemaphoreType.DMA,
            pltpu.SemaphoreType.DMA,
            pltpu.SemaphoreType.DMA,
        ],
        compiler_params=pltpu.CompilerParams(use_tc_tiling_on_sc=False),
    )
    def body(idx_hbm, table_hbm, out_hbm, idx_all, rows0, rows1,
             sg0, sg1, so0, so1):
        rows = (rows0, rows1)
        sem_g = (sg0, sg1)
        sem_o = (so0, so1)
        wid = lax.axis_index("s") * NC + lax.axis_index("c")
        row_base = wid * PER_W

        pltpu.sync_copy(idx_hbm.at[pl.ds(wid * IDX_ROWS_W, IDX_ROWS_W)],
                        idx_all)

        def fire(g, slot):
            for j in range(N_G):
                pltpu.async_copy(table_hbm.at[idx_all.at[g * N_G + j]],
                                 rows[slot].at[pl.ds(j * G, G)],
                                 sem_g[slot])

        def drain_gathers(slot):
            for j in range(N_G):
                pltpu.make_async_copy(table_hbm.at[idx_all.at[0]],
                                      rows[slot].at[pl.ds(j * G, G)],
                                      sem_g[slot]).wait()

        def wait_store(slot):
            pltpu.make_async_copy(rows[slot],
                                  out_hbm.at[pl.ds(row_base, CHUNK)],
                                  sem_o[slot]).wait()

        fire(0, 0)

        def it(i, carry):
            for k in range(2):
                g = 2 * i + k
                slot = k
                nxt = 1 - k

                @pl.when(g >= 1)
                def _():
                    wait_store(nxt)

                @pl.when(g + 1 < N_IT)
                def _():
                    fire(g + 1, nxt)

                drain_gathers(slot)
                pltpu.async_copy(rows[slot],
                                 out_hbm.at[pl.ds(row_base + g * CHUNK,
                                                  CHUNK)],
                                 sem_o[slot])
            return carry

        lax.fori_loop(0, N_IT // 2, it, 0)
        wait_store((N_IT - 1) % 2)

    return body


_gather_kernel = _build()


def kernel(input_seq, weights):
    idx = input_seq.reshape(TOTAL // G, G)
    out = _gather_kernel(idx, weights)
    return out.reshape(BATCH, HIST, D)


# speedup vs baseline: 1.4996x; 1.0013x over previous
import functools

import jax
import jax.numpy as jnp
from jax import lax
from jax.experimental import pallas as pl
from jax.experimental.pallas import tpu as pltpu
from jax.experimental.pallas import tpu_sc as plsc

NUM_EMB = 1000000
D = 32
BATCH = 4096
HIST = 200

NC = 2
NS = 16
NW = NC * NS
ROWS_W = BATCH // NW

R = 4
N_IT = ROWS_W // R


def _build():
    mesh = plsc.VectorSubcoreMesh(core_axis_name="c", subcore_axis_name="s")

    @functools.partial(
        pl.kernel,
        mesh=mesh,
        out_type=jax.ShapeDtypeStruct((BATCH, HIST, D), jnp.float32),
        scratch_types=[
            pltpu.VMEM((ROWS_W, HIST), jnp.int32),
            pltpu.VMEM((R, HIST, D), jnp.float32),
            pltpu.VMEM((R, HIST, D), jnp.float32),
            pltpu.SemaphoreType.DMA,
            pltpu.SemaphoreType.DMA,
            pltpu.SemaphoreType.DMA,
            pltpu.SemaphoreType.DMA,
        ],
        compiler_params=pltpu.CompilerParams(use_tc_tiling_on_sc=False),
    )
    def body(idx_hbm, table_hbm, out_hbm, idx_all, rows0, rows1,
             sg0, sg1, so0, so1):
        rows = (rows0, rows1)
        sem_g = (sg0, sg1)
        sem_o = (so0, so1)
        wid = lax.axis_index("s") * NC + lax.axis_index("c")
        b_base = wid * ROWS_W

        pltpu.sync_copy(idx_hbm.at[pl.ds(b_base, ROWS_W)], idx_all)

        def fire(g, slot):
            for r in range(R):
                pltpu.async_copy(table_hbm.at[idx_all.at[g * R + r]],
                                 rows[slot].at[r],
                                 sem_g[slot])

        def drain_gathers(slot):
            for r in range(R):
                pltpu.make_async_copy(table_hbm.at[idx_all.at[0]],
                                      rows[slot].at[r],
                                      sem_g[slot]).wait()

        def wait_store(slot):
            pltpu.make_async_copy(rows[slot],
                                  out_hbm.at[pl.ds(b_base, R)],
                                  sem_o[slot]).wait()

        fire(0, 0)

        def it(i, carry):
            for k in range(2):
                g = 2 * i + k
                slot = k
                nxt = 1 - k

                @pl.when(g >= 1)
                def _():
                    wait_store(nxt)

                @pl.when(g + 1 < N_IT)
                def _():
                    fire(g + 1, nxt)

                drain_gathers(slot)
                pltpu.async_copy(rows[slot],
                                 out_hbm.at[pl.ds(b_base + g * R, R)],
                                 sem_o[slot])
            return carry

        lax.fori_loop(0, N_IT // 2, it, 0)
        wait_store((N_IT - 1) % 2)

    return body


_gather_kernel = _build()


def kernel(input_seq, weights):
    return _gather_kernel(input_seq, weights)
